# baseline (speedup 1.0000x reference)
import jax
import jax.numpy as jnp
from jax import lax
from jax.experimental import pallas as pl
from jax.experimental.pallas import tpu as pltpu
from jax.experimental.pallas import tpu_sc as plsc


N_NODES = 10000
N_EDGES = 160000
HEADS = 4
CHANNELS = 256
HALF = 128

NC = 2
NS = 16
LANES = 16

EPT = 10240
E_PAD = EPT * NS
K = 64
NCH = EPT // K
NBUF = 4
NPAD = 10240
NPT = NPAD // NS
CLAMP = 75.0


def _tc_table_kernel(x_ref, w_ref, f_ref, xsb_ref):
    xh = x_ref[...]
    a = jnp.dot(xh, w_ref[...], preferred_element_type=jnp.float32,
                precision=lax.Precision.HIGHEST)
    a = jnp.maximum(a, 0.2 * a)
    f_ref[...] = jnp.exp(jnp.minimum(a, CLAMP))
    xsb_ref[...] = jnp.stack([xh[:, :HALF], xh[:, HALF:]],
                             axis=0).astype(jnp.bfloat16)


def _node_tables(x, att):
    w = jnp.zeros((CHANNELS, HEADS), dtype=jnp.float32)
    w = w.at[0:HALF, 0].set(att[0]).at[HALF:CHANNELS, 1].set(att[1])
    w = w.at[0:HALF, 2].set(att[2]).at[HALF:CHANNELS, 3].set(att[3])
    blk = 2000
    return pl.pallas_call(
        _tc_table_kernel,
        grid=(N_NODES // blk,),
        in_specs=[
            pl.BlockSpec((blk, CHANNELS), lambda i: (i, 0)),
            pl.BlockSpec((CHANNELS, HEADS), lambda i: (0, 0)),
        ],
        out_specs=[
            pl.BlockSpec((blk, HEADS), lambda i: (i, 0)),
            pl.BlockSpec((2, blk, HALF), lambda i: (0, i, 0)),
        ],
        out_shape=[
            jax.ShapeDtypeStruct((N_NODES, HEADS), jnp.float32),
            jax.ShapeDtypeStruct((2, N_NODES, HALF), jnp.bfloat16),
        ],
    )(x, w)


def _sc_body(xs_hbm, tabs_hbm, src_hbm, dst_hbm, zrow_hbm, zcol_hbm, out_hbm,
             sidx, didx, gidx, fidx, fbuf0, fbuf1, gbuf, sbuf, abuf0, abuf1,
             out_acc, asum0, asum1, tabs_s0, tabs_s1,
             is0, is1, is2, is3, fs0, fs1, fs2, fs3, gs0, gs1, gs2, gs3,
             ss0, ss1, ss2, ss3, as0, as1, as2, as3):
    c = lax.axis_index("c")
    s = lax.axis_index("s")
    isem = (is0, is1, is2, is3)
    fsem = (fs0, fs1, fs2, fs3)
    gsem = (gs0, gs1, gs2, gs3)
    ssem = (ss0, ss1, ss2, ss3)
    asem = (as0, as1, as2, as3)

    nbase = s * NPT
    pltpu.sync_copy(zrow_hbm, out_acc.at[pl.ds(nbase, NPT)])
    pltpu.sync_copy(zcol_hbm, asum0.at[pl.ds(nbase, NPT)])
    pltpu.sync_copy(zcol_hbm, asum1.at[pl.ds(nbase, NPT)])
    pltpu.sync_copy(tabs_hbm.at[c, 0, pl.ds(nbase, NPT)],
                    tabs_s0.at[pl.ds(nbase, NPT)])
    pltpu.sync_copy(tabs_hbm.at[c, 1, pl.ds(nbase, NPT)],
                    tabs_s1.at[pl.ds(nbase, NPT)])
    plsc.subcore_barrier()

    e0 = s * EPT
    coff = c * N_NODES

    def start_idx(chv, b):
        off = e0 + chv * K
        pltpu.async_copy(src_hbm.at[pl.ds(off, K)], sidx.at[b], isem[b])
        pltpu.async_copy(dst_hbm.at[pl.ds(off, K)], didx.at[b], isem[b])

    def wait_idx(b):
        pltpu.make_async_copy(src_hbm.at[pl.ds(0, K)], sidx.at[b],
                              isem[b]).wait()
        pltpu.make_async_copy(dst_hbm.at[pl.ds(0, K)], didx.at[b],
                              isem[b]).wait()

    def prep_gidx(b):
        for g in range(K // LANES):
            sl = pl.ds(g * LANES, LANES)
            sg = sidx[b, sl]
            gidx[b, sl] = jnp.minimum(sg, N_NODES - 1) + coff
            fidx[b, sl] = jnp.where(c == 0, sg, didx[b, sl])

    def start_fgather(b):
        pltpu.async_copy(tabs_s0.at[fidx.at[b]], fbuf0.at[b], fsem[b])
        pltpu.async_copy(tabs_s1.at[fidx.at[b]], fbuf1.at[b], fsem[b])

    def wait_fgather(b):
        pltpu.make_async_copy(tabs_s0.at[fidx.at[b]], fbuf0.at[b],
                              fsem[b]).wait()
        pltpu.make_async_copy(tabs_s1.at[fidx.at[b]], fbuf1.at[b],
                              fsem[b]).wait()

    def start_asum(b):
        pltpu.async_copy(fbuf0.at[b], asum0.at[didx.at[b]], asem[b], add=True)
        pltpu.async_copy(fbuf1.at[b], asum1.at[didx.at[b]], asem[b], add=True)

    def wait_asum(b):
        pltpu.make_async_copy(fbuf0.at[b], asum0.at[didx.at[b]],
                              asem[b]).wait()
        pltpu.make_async_copy(fbuf1.at[b], asum1.at[didx.at[b]],
                              asem[b]).wait()

    def start_gather(b):
        pltpu.async_copy(xs_hbm.at[gidx.at[b]], gbuf.at[b], gsem[b])

    def wait_gather(b):
        pltpu.make_async_copy(xs_hbm.at[gidx.at[0]], gbuf.at[b],
                              gsem[b]).wait()

    def start_scatter(sb, b):
        pltpu.async_copy(sbuf.at[sb], out_acc.at[didx.at[b]], ssem[sb],
                         add=True)

    def wait_scatter(sb):
        pltpu.make_async_copy(sbuf.at[sb], out_acc.at[didx.at[0]],
                              ssem[sb]).wait()

    iota2 = lax.iota(jnp.int32, LANES) * 2

    def scale(b, sb):
        bv = jnp.full((LANES,), b, jnp.int32)
        sbv = jnp.full((LANES,), sb, jnp.int32)

        @plsc.parallel_loop(0, K, 1, unroll=2)
        def _scale(j):
            jv = jnp.full((LANES,), j, jnp.int32)
            s0 = plsc.load_gather(fbuf0, [bv, jv])
            s1 = plsc.load_gather(fbuf1, [bv, jv])
            for g in range(4):
                w = plsc.bitcast(gbuf[b, j, pl.ds(g * LANES, LANES)],
                                 jnp.bfloat16)
                ev, od = plsc.unpack(w, format=plsc.PackFormat.INTERLEAVED,
                                     preferred_element_type=jnp.float32)
                sc = s0 if g < 2 else s1
                plsc.store_scatter(sbuf, [sbv, jv, iota2 + (g * 32)],
                                   ev * sc)
                plsc.store_scatter(sbuf, [sbv, jv, iota2 + (g * 32 + 1)],
                                   od * sc)

    def prepare(group, ch0):
        for i, b in enumerate(group):
            wait_idx(b)
            prep_gidx(b)
            start_fgather(b)
            start_gather(b)
        for b in group:
            wait_fgather(b)
            start_asum(b)

    def consume(group, ch0, nxt):
        for i, b in enumerate(group):
            wait_gather(b)
            scale(b, i)
            start_scatter(i, b)
        for i, b in enumerate(group):
            wait_scatter(i)
            wait_asum(b)
            if nxt is not None:
                start_idx(nxt + i, b)

    G0 = (0, 1)
    G1 = (2, 3)
    NG = len(G0)

    for b in range(NBUF):
        start_idx(jnp.int32(b), b)
    prepare(G0, jnp.int32(0))

    @pl.loop(0, NCH // NBUF - 1)
    def _main(t):
        cb = t * NBUF
        prepare(G1, cb + NG)
        consume(G0, cb, cb + NBUF)
        prepare(G0, cb + NBUF)
        consume(G1, cb + NG, cb + NBUF + NG)

    lcb = jnp.int32(NCH - NBUF)
    prepare(G1, lcb + NG)
    consume(G0, lcb, None)
    consume(G1, lcb + NG, None)
    plsc.subcore_barrier()

    for k in range(NPT // K):
        n0 = nbase + k * K
        pltpu.sync_copy(out_acc.at[pl.ds(n0, K)], sbuf.at[0])
        pltpu.sync_copy(asum0.at[pl.ds(n0, K)], abuf0)
        pltpu.sync_copy(asum1.at[pl.ds(n0, K)], abuf1)

        @pl.loop(0, K)
        def _norm(j):
            jv = jnp.full((LANES,), j, jnp.int32)
            s0 = 1.0 / jnp.maximum(plsc.load_gather(abuf0, [jv]), 1e-10)
            s1 = 1.0 / jnp.maximum(plsc.load_gather(abuf1, [jv]), 1e-10)
            for r in range(8):
                sl = sbuf[0, j, pl.ds(r * LANES, LANES)]
                sbuf[0, j, pl.ds(r * LANES, LANES)] = sl * (s0 if r < 4 else s1)

        pltpu.sync_copy(sbuf.at[0], out_hbm.at[pl.ds(n0, K), c])


def _sc_aggregate(xs, tabs, srcp, dstp, zrow, zcol):
    mesh = plsc.VectorSubcoreMesh(core_axis_name="c", subcore_axis_name="s")
    return pl.kernel(
        _sc_body,
        out_type=jax.ShapeDtypeStruct((NPAD, NC, HALF), jnp.float32),
        mesh=mesh,
        compiler_params=pltpu.CompilerParams(needs_layout_passes=False,
                                             use_tc_tiling_on_sc=False),
        scratch_types=[
            pltpu.VMEM((NBUF, K), jnp.int32),
            pltpu.VMEM((NBUF, K), jnp.int32),
            pltpu.VMEM((NBUF, K), jnp.int32),
            pltpu.VMEM((NBUF, K), jnp.int32),
            pltpu.VMEM((NBUF, K), jnp.float32),
            pltpu.VMEM((NBUF, K), jnp.float32),
            pltpu.VMEM((NBUF, K, HALF // 2), jnp.int32),
            pltpu.VMEM((2, K, HALF), jnp.float32),
            pltpu.VMEM((K,), jnp.float32),
            pltpu.VMEM((K,), jnp.float32),
            pltpu.VMEM_SHARED((NPAD, HALF), jnp.float32),
            pltpu.VMEM_SHARED((NPAD,), jnp.float32),
            pltpu.VMEM_SHARED((NPAD,), jnp.float32),
            pltpu.VMEM_SHARED((NPAD,), jnp.float32),
            pltpu.VMEM_SHARED((NPAD,), jnp.float32),
        ] + [pltpu.SemaphoreType.DMA] * 20,
    )(xs, tabs, srcp, dstp, zrow, zcol)


def kernel(x, edge_index, att):
    x = x.astype(jnp.float32)
    att = att.astype(jnp.float32)
    src = edge_index[0].astype(jnp.int32)
    dst = edge_index[1].astype(jnp.int32)

    f, xsb = _node_tables(x, att)
    fpad = jnp.pad(f, ((0, NPAD - N_NODES), (0, 0)))
    tabs = fpad.reshape(NPAD, 2, 2).transpose(1, 2, 0)
    xs = lax.bitcast_convert_type(
        xsb.reshape(NC * N_NODES, HALF // 2, 2), jnp.int32)
    pad = E_PAD - N_EDGES
    srcp = jnp.pad(src, (0, pad), constant_values=N_NODES)
    dstp = jnp.pad(dst, (0, pad), constant_values=N_NODES)
    zrow = jnp.zeros((NPT, HALF), jnp.float32)
    zcol = jnp.zeros((NPT,), jnp.float32)

    out3 = _sc_aggregate(xs, tabs, srcp, dstp, zrow, zcol)
    return out3[:N_NODES].reshape(N_NODES, CHANNELS)


# speedup vs baseline: 1.1097x; 1.1097x over previous
import jax
import jax.numpy as jnp
from jax import lax
from jax.experimental import pallas as pl
from jax.experimental.pallas import tpu as pltpu
from jax.experimental.pallas import tpu_sc as plsc


N_NODES = 10000
N_EDGES = 160000
HEADS = 4
CHANNELS = 256
HALF = 128

NC = 2
NS = 16
LANES = 16

EPT = 10240
E_PAD = EPT * NS
K = 64
NCH = EPT // K
NBUF = 4
NPAD = 10240
NPT = NPAD // NS
CLAMP = 75.0


def _tc_table_kernel(x_ref, w_ref, f_ref, xsb_ref):
    xh = x_ref[...]
    a = jnp.dot(xh, w_ref[...], preferred_element_type=jnp.float32,
                precision=lax.Precision.HIGHEST)
    a = jnp.maximum(a, 0.2 * a)
    f_ref[...] = jnp.exp(jnp.minimum(a, CLAMP))
    xsb_ref[...] = jnp.stack([xh[:, :HALF], xh[:, HALF:]],
                             axis=0).astype(jnp.bfloat16)


def _node_tables(x, att):
    w = jnp.zeros((CHANNELS, HEADS), dtype=jnp.float32)
    w = w.at[0:HALF, 0].set(att[0]).at[HALF:CHANNELS, 1].set(att[1])
    w = w.at[0:HALF, 2].set(att[2]).at[HALF:CHANNELS, 3].set(att[3])
    blk = 2000
    return pl.pallas_call(
        _tc_table_kernel,
        grid=(N_NODES // blk,),
        in_specs=[
            pl.BlockSpec((blk, CHANNELS), lambda i: (i, 0)),
            pl.BlockSpec((CHANNELS, HEADS), lambda i: (0, 0)),
        ],
        out_specs=[
            pl.BlockSpec((blk, HEADS), lambda i: (i, 0)),
            pl.BlockSpec((2, blk, HALF), lambda i: (0, i, 0)),
        ],
        out_shape=[
            jax.ShapeDtypeStruct((N_NODES, HEADS), jnp.float32),
            jax.ShapeDtypeStruct((2, N_NODES, HALF), jnp.bfloat16),
        ],
    )(x, w)


def _sc_body(xs_hbm, tabs_hbm, src_hbm, dst_hbm, zrow_hbm, zcol_hbm, out_hbm,
             sidx, didx, gidx, fidx, fbuf0, fbuf1, gbuf, sbuf, abuf0, abuf1,
             out_acc, asum0, asum1, tabs_s0, tabs_s1,
             is0, is1, is2, is3, fs0, fs1, fs2, fs3, gs0, gs1, gs2, gs3,
             ss0, ss1, ss2, ss3, as0, as1, as2, as3):
    c = lax.axis_index("c")
    s = lax.axis_index("s")
    isem = (is0, is1, is2, is3)
    fsem = (fs0, fs1, fs2, fs3)
    gsem = (gs0, gs1, gs2, gs3)
    ssem = (ss0, ss1, ss2, ss3)
    asem = (as0, as1, as2, as3)

    nbase = s * NPT
    pltpu.sync_copy(zrow_hbm, out_acc.at[pl.ds(nbase, NPT)])
    pltpu.sync_copy(zcol_hbm, asum0.at[pl.ds(nbase, NPT)])
    pltpu.sync_copy(zcol_hbm, asum1.at[pl.ds(nbase, NPT)])
    pltpu.sync_copy(tabs_hbm.at[c, 0, pl.ds(nbase, NPT)],
                    tabs_s0.at[pl.ds(nbase, NPT)])
    pltpu.sync_copy(tabs_hbm.at[c, 1, pl.ds(nbase, NPT)],
                    tabs_s1.at[pl.ds(nbase, NPT)])
    plsc.subcore_barrier()

    e0 = s * EPT
    coff = c * N_NODES

    def start_idx(chv, b):
        off = e0 + chv * K
        pltpu.async_copy(src_hbm.at[pl.ds(off, K)], sidx.at[b], isem[b])
        pltpu.async_copy(dst_hbm.at[pl.ds(off, K)], didx.at[b], isem[b])

    def wait_idx(b):
        pltpu.make_async_copy(src_hbm.at[pl.ds(0, K)], sidx.at[b],
                              isem[b]).wait()
        pltpu.make_async_copy(dst_hbm.at[pl.ds(0, K)], didx.at[b],
                              isem[b]).wait()

    def prep_gidx(b):
        for g in range(K // LANES):
            sl = pl.ds(g * LANES, LANES)
            sg = sidx[b, sl]
            gidx[b, sl] = jnp.minimum(sg, N_NODES - 1) + coff
            fidx[b, sl] = jnp.where(c == 0, sg, didx[b, sl])

    def start_fgather(b):
        pltpu.async_copy(tabs_s0.at[fidx.at[b]], fbuf0.at[b], fsem[b])
        pltpu.async_copy(tabs_s1.at[fidx.at[b]], fbuf1.at[b], fsem[b])

    def wait_fgather(b):
        pltpu.make_async_copy(tabs_s0.at[fidx.at[b]], fbuf0.at[b],
                              fsem[b]).wait()
        pltpu.make_async_copy(tabs_s1.at[fidx.at[b]], fbuf1.at[b],
                              fsem[b]).wait()

    def start_asum(b):
        pltpu.async_copy(fbuf0.at[b], asum0.at[didx.at[b]], asem[b], add=True)
        pltpu.async_copy(fbuf1.at[b], asum1.at[didx.at[b]], asem[b], add=True)

    def wait_asum(b):
        pltpu.make_async_copy(fbuf0.at[b], asum0.at[didx.at[b]],
                              asem[b]).wait()
        pltpu.make_async_copy(fbuf1.at[b], asum1.at[didx.at[b]],
                              asem[b]).wait()

    def start_gather(b):
        pltpu.async_copy(xs_hbm.at[gidx.at[b]], gbuf.at[b], gsem[b])

    def wait_gather(b):
        pltpu.make_async_copy(xs_hbm.at[gidx.at[0]], gbuf.at[b],
                              gsem[b]).wait()

    def start_scatter(sb, b):
        pltpu.async_copy(sbuf.at[sb], out_acc.at[didx.at[b]], ssem[sb],
                         add=True)

    def wait_scatter(sb):
        pltpu.make_async_copy(sbuf.at[sb], out_acc.at[didx.at[0]],
                              ssem[sb]).wait()

    iota2 = lax.iota(jnp.int32, LANES) * 2

    def scale(b, sb):
        bv = jnp.full((LANES,), b, jnp.int32)
        sbv = jnp.full((LANES,), sb, jnp.int32)

        @plsc.parallel_loop(0, K, 1, unroll=2)
        def _scale(j):
            jv = jnp.full((LANES,), j, jnp.int32)
            s0 = plsc.load_gather(fbuf0, [bv, jv])
            s1 = plsc.load_gather(fbuf1, [bv, jv])
            for g in range(4):
                w = plsc.bitcast(gbuf[b, j, pl.ds(g * LANES, LANES)],
                                 jnp.bfloat16)
                ev, od = plsc.unpack(w, format=plsc.PackFormat.INTERLEAVED,
                                     preferred_element_type=jnp.float32)
                sc = s0 if g < 2 else s1
                plsc.store_scatter(sbuf, [sbv, jv, iota2 + (g * 32)],
                                   ev * sc)
                plsc.store_scatter(sbuf, [sbv, jv, iota2 + (g * 32 + 1)],
                                   od * sc)

    def prepare(group, ch0):
        for i, b in enumerate(group):
            wait_idx(b)
            prep_gidx(b)
            start_fgather(b)
            start_gather(b)
        for b in group:
            wait_fgather(b)
            start_asum(b)

    def consume(group, ch0, nxt):
        for i, b in enumerate(group):
            wait_gather(b)
            scale(b, i)
            start_scatter(i, b)
        for i, b in enumerate(group):
            wait_scatter(i)
            wait_asum(b)
            if nxt is not None:
                start_idx(nxt + i, b)

    G0 = (0, 1)
    G1 = (2, 3)
    NG = len(G0)

    for b in range(NBUF):
        start_idx(jnp.int32(b), b)
    prepare(G0, jnp.int32(0))

    @pl.loop(0, NCH // NBUF - 1)
    def _main(t):
        cb = t * NBUF
        prepare(G1, cb + NG)
        consume(G0, cb, cb + NBUF)
        prepare(G0, cb + NBUF)
        consume(G1, cb + NG, cb + NBUF + NG)

    lcb = jnp.int32(NCH - NBUF)
    prepare(G1, lcb + NG)
    consume(G0, lcb, None)
    consume(G1, lcb + NG, None)
    plsc.subcore_barrier()

    obase = c * NPAD + nbase
    for k in range(NPT // K):
        n0 = nbase + k * K
        pltpu.sync_copy(out_acc.at[pl.ds(n0, K)], sbuf.at[0])
        pltpu.sync_copy(asum0.at[pl.ds(n0, K)], abuf0)
        pltpu.sync_copy(asum1.at[pl.ds(n0, K)], abuf1)

        @pl.loop(0, K)
        def _norm(j):
            jv = jnp.full((LANES,), j, jnp.int32)
            s0 = 1.0 / jnp.maximum(plsc.load_gather(abuf0, [jv]), 1e-10)
            s1 = 1.0 / jnp.maximum(plsc.load_gather(abuf1, [jv]), 1e-10)
            for r in range(8):
                sl = sbuf[0, j, pl.ds(r * LANES, LANES)]
                sbuf[0, j, pl.ds(r * LANES, LANES)] = sl * (s0 if r < 4 else s1)

        pltpu.sync_copy(sbuf.at[0], out_hbm.at[pl.ds(obase + k * K, K)])


def _sc_aggregate(xs, tabs, srcp, dstp, zrow, zcol):
    mesh = plsc.VectorSubcoreMesh(core_axis_name="c", subcore_axis_name="s")
    return pl.kernel(
        _sc_body,
        out_type=jax.ShapeDtypeStruct((NC * NPAD, HALF), jnp.float32),
        mesh=mesh,
        compiler_params=pltpu.CompilerParams(needs_layout_passes=False,
                                             use_tc_tiling_on_sc=False),
        scratch_types=[
            pltpu.VMEM((NBUF, K), jnp.int32),
            pltpu.VMEM((NBUF, K), jnp.int32),
            pltpu.VMEM((NBUF, K), jnp.int32),
            pltpu.VMEM((NBUF, K), jnp.int32),
            pltpu.VMEM((NBUF, K), jnp.float32),
            pltpu.VMEM((NBUF, K), jnp.float32),
            pltpu.VMEM((NBUF, K, HALF // 2), jnp.int32),
            pltpu.VMEM((2, K, HALF), jnp.float32),
            pltpu.VMEM((K,), jnp.float32),
            pltpu.VMEM((K,), jnp.float32),
            pltpu.VMEM_SHARED((NPAD, HALF), jnp.float32),
            pltpu.VMEM_SHARED((NPAD,), jnp.float32),
            pltpu.VMEM_SHARED((NPAD,), jnp.float32),
            pltpu.VMEM_SHARED((NPAD,), jnp.float32),
            pltpu.VMEM_SHARED((NPAD,), jnp.float32),
        ] + [pltpu.SemaphoreType.DMA] * 20,
    )(xs, tabs, srcp, dstp, zrow, zcol)


def kernel(x, edge_index, att):
    x = x.astype(jnp.float32)
    att = att.astype(jnp.float32)
    src = edge_index[0].astype(jnp.int32)
    dst = edge_index[1].astype(jnp.int32)

    f, xsb = _node_tables(x, att)
    fpad = jnp.pad(f, ((0, NPAD - N_NODES), (0, 0)))
    tabs = fpad.reshape(NPAD, 2, 2).transpose(1, 2, 0)
    xs = lax.bitcast_convert_type(
        xsb.reshape(NC * N_NODES, HALF // 2, 2), jnp.int32)
    pad = E_PAD - N_EDGES
    srcp = jnp.pad(src, (0, pad), constant_values=N_NODES)
    dstp = jnp.pad(dst, (0, pad), constant_values=N_NODES)
    zrow = jnp.zeros((NPT, HALF), jnp.float32)
    zcol = jnp.zeros((NPT,), jnp.float32)

    out2 = _sc_aggregate(xs, tabs, srcp, dstp, zrow, zcol)
    return (out2.reshape(NC, NPAD, HALF)[:, :N_NODES, :].transpose(1, 0, 2)
            .reshape(N_NODES, CHANNELS))


# speedup vs baseline: 1.1450x; 1.0318x over previous
import jax
import jax.numpy as jnp
from jax import lax
from jax.experimental import pallas as pl
from jax.experimental.pallas import tpu as pltpu
from jax.experimental.pallas import tpu_sc as plsc


N_NODES = 10000
N_EDGES = 160000
HEADS = 4
CHANNELS = 256
HALF = 128

NC = 2
NS = 16
LANES = 16

EPT = 10240
E_PAD = EPT * NS
K = 64
NCH = EPT // K
NBUF = 4
NPAD = 10240
NPT = NPAD // NS
CLAMP = 75.0


def _tc_table_kernel(x_ref, att_ref, f_ref, xsb_ref):
    xh = x_ref[...]
    att = att_ref[...]
    lo, hi = xh[:, :HALF], xh[:, HALF:]
    cols = []
    for h in range(HEADS):
        half = lo if h % 2 == 0 else hi
        cols.append(jnp.sum(half * att[h][None, :], axis=1, keepdims=True))
    a = jnp.concatenate(cols, axis=1)
    a = jnp.maximum(a, 0.2 * a)
    f_ref[...] = jnp.exp(jnp.minimum(a, CLAMP))
    xsb_ref[...] = jnp.stack([lo, hi], axis=0).astype(jnp.bfloat16)


def _node_tables(x, att):
    blk = 2000
    return pl.pallas_call(
        _tc_table_kernel,
        grid=(N_NODES // blk,),
        in_specs=[
            pl.BlockSpec((blk, CHANNELS), lambda i: (i, 0)),
            pl.BlockSpec((HEADS, HALF), lambda i: (0, 0)),
        ],
        out_specs=[
            pl.BlockSpec((blk, HEADS), lambda i: (i, 0)),
            pl.BlockSpec((2, blk, HALF), lambda i: (0, i, 0)),
        ],
        out_shape=[
            jax.ShapeDtypeStruct((N_NODES, HEADS), jnp.float32),
            jax.ShapeDtypeStruct((2, N_NODES, HALF), jnp.bfloat16),
        ],
    )(x, att)


def _sc_body(xs_hbm, tabs_hbm, src_hbm, dst_hbm, zrow_hbm, zcol_hbm, out_hbm,
             sidx, didx, gidx, fidx, fbuf0, fbuf1, gbuf, sbuf, abuf0, abuf1,
             out_acc, asum0, asum1, tabs_s0, tabs_s1,
             is0, is1, is2, is3, fs0, fs1, fs2, fs3, gs0, gs1, gs2, gs3,
             ss0, ss1, ss2, ss3, as0, as1, as2, as3):
    c = lax.axis_index("c")
    s = lax.axis_index("s")
    isem = (is0, is1, is2, is3)
    fsem = (fs0, fs1, fs2, fs3)
    gsem = (gs0, gs1, gs2, gs3)
    ssem = (ss0, ss1, ss2, ss3)
    asem = (as0, as1, as2, as3)

    nbase = s * NPT
    pltpu.sync_copy(zrow_hbm, out_acc.at[pl.ds(nbase, NPT)])
    pltpu.sync_copy(zcol_hbm, asum0.at[pl.ds(nbase, NPT)])
    pltpu.sync_copy(zcol_hbm, asum1.at[pl.ds(nbase, NPT)])
    pltpu.sync_copy(tabs_hbm.at[c, 0, pl.ds(nbase, NPT)],
                    tabs_s0.at[pl.ds(nbase, NPT)])
    pltpu.sync_copy(tabs_hbm.at[c, 1, pl.ds(nbase, NPT)],
                    tabs_s1.at[pl.ds(nbase, NPT)])
    plsc.subcore_barrier()

    e0 = s * EPT
    coff = c * N_NODES

    def start_idx(chv, b):
        off = e0 + chv * K
        pltpu.async_copy(src_hbm.at[pl.ds(off, K)], sidx.at[b], isem[b])
        pltpu.async_copy(dst_hbm.at[pl.ds(off, K)], didx.at[b], isem[b])

    def wait_idx(b):
        pltpu.make_async_copy(src_hbm.at[pl.ds(0, K)], sidx.at[b],
                              isem[b]).wait()
        pltpu.make_async_copy(dst_hbm.at[pl.ds(0, K)], didx.at[b],
                              isem[b]).wait()

    def prep_gidx(b):
        for g in range(K // LANES):
            sl = pl.ds(g * LANES, LANES)
            sg = sidx[b, sl]
            gidx[b, sl] = jnp.minimum(sg, N_NODES - 1) + coff
            fidx[b, sl] = jnp.where(c == 0, sg, didx[b, sl])

    def start_fgather(b):
        pltpu.async_copy(tabs_s0.at[fidx.at[b]], fbuf0.at[b], fsem[b])
        pltpu.async_copy(tabs_s1.at[fidx.at[b]], fbuf1.at[b], fsem[b])

    def wait_fgather(b):
        pltpu.make_async_copy(tabs_s0.at[fidx.at[b]], fbuf0.at[b],
                              fsem[b]).wait()
        pltpu.make_async_copy(tabs_s1.at[fidx.at[b]], fbuf1.at[b],
                              fsem[b]).wait()

    def start_asum(b):
        pltpu.async_copy(fbuf0.at[b], asum0.at[didx.at[b]], asem[b], add=True)
        pltpu.async_copy(fbuf1.at[b], asum1.at[didx.at[b]], asem[b], add=True)

    def wait_asum(b):
        pltpu.make_async_copy(fbuf0.at[b], asum0.at[didx.at[b]],
                              asem[b]).wait()
        pltpu.make_async_copy(fbuf1.at[b], asum1.at[didx.at[b]],
                              asem[b]).wait()

    def start_gather(b):
        pltpu.async_copy(xs_hbm.at[gidx.at[b]], gbuf.at[b], gsem[b])

    def wait_gather(b):
        pltpu.make_async_copy(xs_hbm.at[gidx.at[0]], gbuf.at[b],
                              gsem[b]).wait()

    def start_scatter(sb, b):
        pltpu.async_copy(sbuf.at[sb], out_acc.at[didx.at[b]], ssem[sb],
                         add=True)

    def wait_scatter(sb):
        pltpu.make_async_copy(sbuf.at[sb], out_acc.at[didx.at[0]],
                              ssem[sb]).wait()

    iota2 = lax.iota(jnp.int32, LANES) * 2

    def scale(b, sb):
        bv = jnp.full((LANES,), b, jnp.int32)
        sbv = jnp.full((LANES,), sb, jnp.int32)

        @plsc.parallel_loop(0, K, 1, unroll=2)
        def _scale(j):
            jv = jnp.full((LANES,), j, jnp.int32)
            s0 = plsc.load_gather(fbuf0, [bv, jv])
            s1 = plsc.load_gather(fbuf1, [bv, jv])
            for g in range(4):
                w = plsc.bitcast(gbuf[b, j, pl.ds(g * LANES, LANES)],
                                 jnp.bfloat16)
                ev, od = plsc.unpack(w, format=plsc.PackFormat.INTERLEAVED,
                                     preferred_element_type=jnp.float32)
                sc = s0 if g < 2 else s1
                plsc.store_scatter(sbuf, [sbv, jv, iota2 + (g * 32)],
                                   ev * sc)
                plsc.store_scatter(sbuf, [sbv, jv, iota2 + (g * 32 + 1)],
                                   od * sc)

    def prepare(group, ch0):
        for i, b in enumerate(group):
            wait_idx(b)
            prep_gidx(b)
            start_fgather(b)
            start_gather(b)
        for b in group:
            wait_fgather(b)
            start_asum(b)

    def consume(group, ch0, nxt):
        for i, b in enumerate(group):
            wait_gather(b)
            scale(b, i)
            start_scatter(i, b)
        for i, b in enumerate(group):
            wait_scatter(i)
            wait_asum(b)
            if nxt is not None:
                start_idx(nxt + i, b)

    G0 = (0, 1)
    G1 = (2, 3)
    NG = len(G0)

    for b in range(NBUF):
        start_idx(jnp.int32(b), b)
    prepare(G0, jnp.int32(0))

    @pl.loop(0, NCH // NBUF - 1)
    def _main(t):
        cb = t * NBUF
        prepare(G1, cb + NG)
        consume(G0, cb, cb + NBUF)
        prepare(G0, cb + NBUF)
        consume(G1, cb + NG, cb + NBUF + NG)

    lcb = jnp.int32(NCH - NBUF)
    prepare(G1, lcb + NG)
    consume(G0, lcb, None)
    consume(G1, lcb + NG, None)
    plsc.subcore_barrier()

    obase = c * NPAD + nbase
    for k in range(NPT // K):
        n0 = nbase + k * K
        pltpu.sync_copy(out_acc.at[pl.ds(n0, K)], sbuf.at[0])
        pltpu.sync_copy(asum0.at[pl.ds(n0, K)], abuf0)
        pltpu.sync_copy(asum1.at[pl.ds(n0, K)], abuf1)

        @pl.loop(0, K)
        def _norm(j):
            jv = jnp.full((LANES,), j, jnp.int32)
            s0 = 1.0 / jnp.maximum(plsc.load_gather(abuf0, [jv]), 1e-10)
            s1 = 1.0 / jnp.maximum(plsc.load_gather(abuf1, [jv]), 1e-10)
            for r in range(8):
                sl = sbuf[0, j, pl.ds(r * LANES, LANES)]
                sbuf[0, j, pl.ds(r * LANES, LANES)] = sl * (s0 if r < 4 else s1)

        pltpu.sync_copy(sbuf.at[0], out_hbm.at[pl.ds(obase + k * K, K)])


def _sc_aggregate(xs, tabs, srcp, dstp, zrow, zcol):
    mesh = plsc.VectorSubcoreMesh(core_axis_name="c", subcore_axis_name="s")
    return pl.kernel(
        _sc_body,
        out_type=jax.ShapeDtypeStruct((NC * NPAD, HALF), jnp.float32),
        mesh=mesh,
        compiler_params=pltpu.CompilerParams(needs_layout_passes=False,
                                             use_tc_tiling_on_sc=False),
        scratch_types=[
            pltpu.VMEM((NBUF, K), jnp.int32),
            pltpu.VMEM((NBUF, K), jnp.int32),
            pltpu.VMEM((NBUF, K), jnp.int32),
            pltpu.VMEM((NBUF, K), jnp.int32),
            pltpu.VMEM((NBUF, K), jnp.float32),
            pltpu.VMEM((NBUF, K), jnp.float32),
            pltpu.VMEM((NBUF, K, HALF // 2), jnp.int32),
            pltpu.VMEM((2, K, HALF), jnp.float32),
            pltpu.VMEM((K,), jnp.float32),
            pltpu.VMEM((K,), jnp.float32),
            pltpu.VMEM_SHARED((NPAD, HALF), jnp.float32),
            pltpu.VMEM_SHARED((NPAD,), jnp.float32),
            pltpu.VMEM_SHARED((NPAD,), jnp.float32),
            pltpu.VMEM_SHARED((NPAD,), jnp.float32),
            pltpu.VMEM_SHARED((NPAD,), jnp.float32),
        ] + [pltpu.SemaphoreType.DMA] * 20,
    )(xs, tabs, srcp, dstp, zrow, zcol)


def kernel(x, edge_index, att):
    x = x.astype(jnp.float32)
    att = att.astype(jnp.float32)
    src = edge_index[0].astype(jnp.int32)
    dst = edge_index[1].astype(jnp.int32)

    f, xsb = _node_tables(x, att)
    fpad = jnp.pad(f, ((0, NPAD - N_NODES), (0, 0)))
    tabs = fpad.reshape(NPAD, 2, 2).transpose(1, 2, 0)
    xs = lax.bitcast_convert_type(
        xsb.reshape(NC * N_NODES, HALF // 2, 2), jnp.int32)
    pad = E_PAD - N_EDGES
    srcp = jnp.pad(src, (0, pad), constant_values=N_NODES)
    dstp = jnp.pad(dst, (0, pad), constant_values=N_NODES)
    zrow = jnp.zeros((NPT, HALF), jnp.float32)
    zcol = jnp.zeros((NPT,), jnp.float32)

    out2 = _sc_aggregate(xs, tabs, srcp, dstp, zrow, zcol)
    return (out2.reshape(NC, NPAD, HALF)[:, :N_NODES, :].transpose(1, 0, 2)
            .reshape(N_NODES, CHANNELS))


# speedup vs baseline: 1.1465x; 1.0013x over previous
import jax
import jax.numpy as jnp
from jax import lax
from jax.experimental import pallas as pl
from jax.experimental.pallas import tpu as pltpu
from jax.experimental.pallas import tpu_sc as plsc


N_NODES = 10000
N_EDGES = 160000
HEADS = 4
CHANNELS = 256
HALF = 128

NC = 2
NS = 16
LANES = 16

EPT = 10240
E_PAD = EPT * NS
K = 64
NCH = EPT // K
NBUF = 4
NPAD = 10240
NPT = NPAD // NS
CLAMP = 75.0


def _tc_table_kernel(x_ref, att_ref, f_ref, xsb_ref):
    xh = x_ref[...]
    att = att_ref[...]
    lo, hi = xh[:, :HALF], xh[:, HALF:]
    cols = []
    for h in range(HEADS):
        half = lo if h % 2 == 0 else hi
        cols.append(jnp.sum(half * att[h][None, :], axis=1, keepdims=True))
    a = jnp.concatenate(cols, axis=1)
    a = jnp.maximum(a, 0.2 * a)
    f_ref[...] = jnp.exp(jnp.minimum(a, CLAMP))
    xsb_ref[...] = jnp.stack([lo, hi], axis=0).astype(jnp.bfloat16)


def _node_tables(x, att):
    blk = 2000
    return pl.pallas_call(
        _tc_table_kernel,
        grid=(N_NODES // blk,),
        in_specs=[
            pl.BlockSpec((blk, CHANNELS), lambda i: (i, 0)),
            pl.BlockSpec((HEADS, HALF), lambda i: (0, 0)),
        ],
        out_specs=[
            pl.BlockSpec((blk, HEADS), lambda i: (i, 0)),
            pl.BlockSpec((2, blk, HALF), lambda i: (0, i, 0)),
        ],
        out_shape=[
            jax.ShapeDtypeStruct((N_NODES, HEADS), jnp.float32),
            jax.ShapeDtypeStruct((2, N_NODES, HALF), jnp.bfloat16),
        ],
    )(x, att)


def _sc_body(xs_hbm, tabs_hbm, src_hbm, dst_hbm, zrow_hbm, zcol_hbm, out_hbm,
             sidx, didx, gidx, fidx, fbuf0, fbuf1, gbuf, sbuf, abuf0, abuf1,
             out_acc, asum0, asum1, tabs_s0, tabs_s1,
             is0, is1, is2, is3, fs0, fs1, fs2, fs3, gs0, gs1, gs2, gs3,
             ss0, ss1, ss2, ss3, as0, as1, as2, as3):
    c = lax.axis_index("c")
    s = lax.axis_index("s")
    isem = (is0, is1, is2, is3)
    fsem = (fs0, fs1, fs2, fs3)
    gsem = (gs0, gs1, gs2, gs3)
    ssem = (ss0, ss1, ss2, ss3)
    asem = (as0, as1, as2, as3)

    nbase = s * NPT
    pltpu.sync_copy(zrow_hbm, out_acc.at[pl.ds(nbase, NPT)])
    pltpu.sync_copy(zcol_hbm, asum0.at[pl.ds(nbase, NPT)])
    pltpu.sync_copy(zcol_hbm, asum1.at[pl.ds(nbase, NPT)])
    pltpu.sync_copy(tabs_hbm.at[c, 0, pl.ds(nbase, NPT)],
                    tabs_s0.at[pl.ds(nbase, NPT)])
    pltpu.sync_copy(tabs_hbm.at[c, 1, pl.ds(nbase, NPT)],
                    tabs_s1.at[pl.ds(nbase, NPT)])
    plsc.subcore_barrier()

    e0 = s * EPT
    coff = c * N_NODES

    def start_idx(chv, b):
        off = e0 + chv * K
        pltpu.async_copy(src_hbm.at[pl.ds(off, K)], sidx.at[b], isem[b])
        pltpu.async_copy(dst_hbm.at[pl.ds(off, K)], didx.at[b], isem[b])

    def wait_idx(b):
        pltpu.make_async_copy(src_hbm.at[pl.ds(0, K)], sidx.at[b],
                              isem[b]).wait()
        pltpu.make_async_copy(dst_hbm.at[pl.ds(0, K)], didx.at[b],
                              isem[b]).wait()

    def prep_gidx(b):
        for g in range(K // LANES):
            sl = pl.ds(g * LANES, LANES)
            sg = sidx[b, sl]
            gidx[b, sl] = jnp.minimum(sg, N_NODES - 1) + coff
            fidx[b, sl] = jnp.where(c == 0, sg, didx[b, sl])

    def start_fgather(b):
        pltpu.async_copy(tabs_s0.at[fidx.at[b]], fbuf0.at[b], fsem[b])
        pltpu.async_copy(tabs_s1.at[fidx.at[b]], fbuf1.at[b], fsem[b])

    def wait_fgather(b):
        pltpu.make_async_copy(tabs_s0.at[fidx.at[b]], fbuf0.at[b],
                              fsem[b]).wait()
        pltpu.make_async_copy(tabs_s1.at[fidx.at[b]], fbuf1.at[b],
                              fsem[b]).wait()

    def start_asum(b):
        pltpu.async_copy(fbuf0.at[b], asum0.at[didx.at[b]], asem[b], add=True)
        pltpu.async_copy(fbuf1.at[b], asum1.at[didx.at[b]], asem[b], add=True)

    def wait_asum(b):
        pltpu.make_async_copy(fbuf0.at[b], asum0.at[didx.at[b]],
                              asem[b]).wait()
        pltpu.make_async_copy(fbuf1.at[b], asum1.at[didx.at[b]],
                              asem[b]).wait()

    def start_gather(b):
        pltpu.async_copy(xs_hbm.at[gidx.at[b]], gbuf.at[b], gsem[b])

    def wait_gather(b):
        pltpu.make_async_copy(xs_hbm.at[gidx.at[0]], gbuf.at[b],
                              gsem[b]).wait()

    def start_scatter(sb, b):
        pltpu.async_copy(sbuf.at[sb], out_acc.at[didx.at[b]], ssem[sb],
                         add=True)

    def wait_scatter(sb):
        pltpu.make_async_copy(sbuf.at[sb], out_acc.at[didx.at[0]],
                              ssem[sb]).wait()

    iota2 = lax.iota(jnp.int32, LANES) * 2

    def scale(b, sb):
        bv = jnp.full((LANES,), b, jnp.int32)
        sbv = jnp.full((LANES,), sb, jnp.int32)

        @plsc.parallel_loop(0, K, 1, unroll=4)
        def _scale(j):
            jv = jnp.full((LANES,), j, jnp.int32)
            s0 = plsc.load_gather(fbuf0, [bv, jv])
            s1 = plsc.load_gather(fbuf1, [bv, jv])
            for g in range(4):
                w = plsc.bitcast(gbuf[b, j, pl.ds(g * LANES, LANES)],
                                 jnp.bfloat16)
                ev, od = plsc.unpack(w, format=plsc.PackFormat.INTERLEAVED,
                                     preferred_element_type=jnp.float32)
                sc = s0 if g < 2 else s1
                plsc.store_scatter(sbuf, [sbv, jv, iota2 + (g * 32)],
                                   ev * sc)
                plsc.store_scatter(sbuf, [sbv, jv, iota2 + (g * 32 + 1)],
                                   od * sc)

    def prepare(group, ch0):
        for i, b in enumerate(group):
            wait_idx(b)
            prep_gidx(b)
            start_fgather(b)
            start_gather(b)
        for b in group:
            wait_fgather(b)
            start_asum(b)

    def consume(group, ch0, nxt):
        for i, b in enumerate(group):
            wait_gather(b)
            scale(b, i)
            start_scatter(i, b)
        for i, b in enumerate(group):
            wait_scatter(i)
            wait_asum(b)
            if nxt is not None:
                start_idx(nxt + i, b)

    G0 = (0, 1)
    G1 = (2, 3)
    NG = len(G0)

    for b in range(NBUF):
        start_idx(jnp.int32(b), b)
    prepare(G0, jnp.int32(0))

    @pl.loop(0, NCH // NBUF - 1)
    def _main(t):
        cb = t * NBUF
        prepare(G1, cb + NG)
        consume(G0, cb, cb + NBUF)
        prepare(G0, cb + NBUF)
        consume(G1, cb + NG, cb + NBUF + NG)

    lcb = jnp.int32(NCH - NBUF)
    prepare(G1, lcb + NG)
    consume(G0, lcb, None)
    consume(G1, lcb + NG, None)
    plsc.subcore_barrier()

    obase = c * NPAD + nbase
    for k in range(NPT // K):
        n0 = nbase + k * K
        pltpu.sync_copy(out_acc.at[pl.ds(n0, K)], sbuf.at[0])
        pltpu.sync_copy(asum0.at[pl.ds(n0, K)], abuf0)
        pltpu.sync_copy(asum1.at[pl.ds(n0, K)], abuf1)

        @pl.loop(0, K)
        def _norm(j):
            jv = jnp.full((LANES,), j, jnp.int32)
            s0 = 1.0 / jnp.maximum(plsc.load_gather(abuf0, [jv]), 1e-10)
            s1 = 1.0 / jnp.maximum(plsc.load_gather(abuf1, [jv]), 1e-10)
            for r in range(8):
                sl = sbuf[0, j, pl.ds(r * LANES, LANES)]
                sbuf[0, j, pl.ds(r * LANES, LANES)] = sl * (s0 if r < 4 else s1)

        pltpu.sync_copy(sbuf.at[0], out_hbm.at[pl.ds(obase + k * K, K)])


def _sc_aggregate(xs, tabs, srcp, dstp, zrow, zcol):
    mesh = plsc.VectorSubcoreMesh(core_axis_name="c", subcore_axis_name="s")
    return pl.kernel(
        _sc_body,
        out_type=jax.ShapeDtypeStruct((NC * NPAD, HALF), jnp.float32),
        mesh=mesh,
        compiler_params=pltpu.CompilerParams(needs_layout_passes=False,
                                             use_tc_tiling_on_sc=False),
        scratch_types=[
            pltpu.VMEM((NBUF, K), jnp.int32),
            pltpu.VMEM((NBUF, K), jnp.int32),
            pltpu.VMEM((NBUF, K), jnp.int32),
            pltpu.VMEM((NBUF, K), jnp.int32),
            pltpu.VMEM((NBUF, K), jnp.float32),
            pltpu.VMEM((NBUF, K), jnp.float32),
            pltpu.VMEM((NBUF, K, HALF // 2), jnp.int32),
            pltpu.VMEM((2, K, HALF), jnp.float32),
            pltpu.VMEM((K,), jnp.float32),
            pltpu.VMEM((K,), jnp.float32),
            pltpu.VMEM_SHARED((NPAD, HALF), jnp.float32),
            pltpu.VMEM_SHARED((NPAD,), jnp.float32),
            pltpu.VMEM_SHARED((NPAD,), jnp.float32),
            pltpu.VMEM_SHARED((NPAD,), jnp.float32),
            pltpu.VMEM_SHARED((NPAD,), jnp.float32),
        ] + [pltpu.SemaphoreType.DMA] * 20,
    )(xs, tabs, srcp, dstp, zrow, zcol)


def kernel(x, edge_index, att):
    x = x.astype(jnp.float32)
    att = att.astype(jnp.float32)
    src = edge_index[0].astype(jnp.int32)
    dst = edge_index[1].astype(jnp.int32)

    f, xsb = _node_tables(x, att)
    fpad = jnp.pad(f, ((0, NPAD - N_NODES), (0, 0)))
    tabs = fpad.reshape(NPAD, 2, 2).transpose(1, 2, 0)
    xs = lax.bitcast_convert_type(
        xsb.reshape(NC * N_NODES, HALF // 2, 2), jnp.int32)
    pad = E_PAD - N_EDGES
    srcp = jnp.pad(src, (0, pad), constant_values=N_NODES)
    dstp = jnp.pad(dst, (0, pad), constant_values=N_NODES)
    zrow = jnp.zeros((NPT, HALF), jnp.float32)
    zcol = jnp.zeros((NPT,), jnp.float32)

    out2 = _sc_aggregate(xs, tabs, srcp, dstp, zrow, zcol)
    return (out2.reshape(NC, NPAD, HALF)[:, :N_NODES, :].transpose(1, 0, 2)
            .reshape(N_NODES, CHANNELS))


# speedup vs baseline: 1.1663x; 1.0172x over previous
import jax
import jax.numpy as jnp
from jax import lax
from jax.experimental import pallas as pl
from jax.experimental.pallas import tpu as pltpu
from jax.experimental.pallas import tpu_sc as plsc


N_NODES = 10000
N_EDGES = 160000
HEADS = 4
CHANNELS = 256
HALF = 128

NC = 2
NS = 16
LANES = 16

EPT = 10240
E_PAD = EPT * NS
K = 64
NCH = EPT // K
NBUF = 4
NPAD = 10240
NPT = NPAD // NS
CLAMP = 75.0


def _tc_table_kernel(x_ref, att_ref, f_ref, xsb_ref):
    xh = x_ref[...]
    att = att_ref[...]
    lo, hi = xh[:, :HALF], xh[:, HALF:]
    cols = []
    for h in range(HEADS):
        half = lo if h % 2 == 0 else hi
        cols.append(jnp.sum(half * att[h][None, :], axis=1, keepdims=True))
    a = jnp.concatenate(cols, axis=1)
    a = jnp.maximum(a, 0.2 * a)
    f_ref[...] = jnp.exp(jnp.minimum(a, CLAMP))
    xsb_ref[...] = jnp.stack([lo, hi], axis=0).astype(jnp.bfloat16)


def _node_tables(x, att):
    blk = 2000
    return pl.pallas_call(
        _tc_table_kernel,
        grid=(N_NODES // blk,),
        in_specs=[
            pl.BlockSpec((blk, CHANNELS), lambda i: (i, 0)),
            pl.BlockSpec((HEADS, HALF), lambda i: (0, 0)),
        ],
        out_specs=[
            pl.BlockSpec((blk, HEADS), lambda i: (i, 0)),
            pl.BlockSpec((2, blk, HALF), lambda i: (0, i, 0)),
        ],
        out_shape=[
            jax.ShapeDtypeStruct((N_NODES, HEADS), jnp.float32),
            jax.ShapeDtypeStruct((2, N_NODES, HALF), jnp.bfloat16),
        ],
    )(x, att)


def _sc_body(xs_hbm, tabs_hbm, src_hbm, dst_hbm, out_hbm,
             sidx, didx, gidx, fidx, fbuf0, fbuf1, gbuf, sbuf, abuf0, abuf1,
             out_acc, asum0, asum1, tabs_s0, tabs_s1,
             is0, is1, is2, is3, fs0, fs1, fs2, fs3, gs0, gs1, gs2, gs3,
             ss0, ss1, ss2, ss3, as0, as1, as2, as3):
    c = lax.axis_index("c")
    s = lax.axis_index("s")
    isem = (is0, is1, is2, is3)
    fsem = (fs0, fs1, fs2, fs3)
    gsem = (gs0, gs1, gs2, gs3)
    ssem = (ss0, ss1, ss2, ss3)
    asem = (as0, as1, as2, as3)

    nbase = s * NPT
    zero16 = jnp.zeros((LANES,), jnp.float32)

    @pl.loop(0, K)
    def _zero_sbuf(j):
        for r in range(8):
            sbuf[0, j, pl.ds(r * LANES, LANES)] = zero16

    for g in range(K // LANES):
        abuf0[pl.ds(g * LANES, LANES)] = zero16
    for k in range(NPT // K):
        pltpu.sync_copy(sbuf.at[0], out_acc.at[pl.ds(nbase + k * K, K)])
        pltpu.sync_copy(abuf0, asum0.at[pl.ds(nbase + k * K, K)])
        pltpu.sync_copy(abuf0, asum1.at[pl.ds(nbase + k * K, K)])
    pltpu.sync_copy(tabs_hbm.at[c, 0, pl.ds(nbase, NPT)],
                    tabs_s0.at[pl.ds(nbase, NPT)])
    pltpu.sync_copy(tabs_hbm.at[c, 1, pl.ds(nbase, NPT)],
                    tabs_s1.at[pl.ds(nbase, NPT)])
    plsc.subcore_barrier()

    e0 = s * EPT
    coff = c * N_NODES

    def start_idx(chv, b):
        off = e0 + chv * K
        pltpu.async_copy(src_hbm.at[pl.ds(off, K)], sidx.at[b], isem[b])
        pltpu.async_copy(dst_hbm.at[pl.ds(off, K)], didx.at[b], isem[b])

    def wait_idx(b):
        pltpu.make_async_copy(src_hbm.at[pl.ds(0, K)], sidx.at[b],
                              isem[b]).wait()
        pltpu.make_async_copy(dst_hbm.at[pl.ds(0, K)], didx.at[b],
                              isem[b]).wait()

    def prep_gidx(b):
        for g in range(K // LANES):
            sl = pl.ds(g * LANES, LANES)
            sg = sidx[b, sl]
            gidx[b, sl] = jnp.minimum(sg, N_NODES - 1) + coff
            fidx[b, sl] = jnp.where(c == 0, sg, didx[b, sl])

    def start_fgather(b):
        pltpu.async_copy(tabs_s0.at[fidx.at[b]], fbuf0.at[b], fsem[b])
        pltpu.async_copy(tabs_s1.at[fidx.at[b]], fbuf1.at[b], fsem[b])

    def wait_fgather(b):
        pltpu.make_async_copy(tabs_s0.at[fidx.at[b]], fbuf0.at[b],
                              fsem[b]).wait()
        pltpu.make_async_copy(tabs_s1.at[fidx.at[b]], fbuf1.at[b],
                              fsem[b]).wait()

    def start_asum(b):
        pltpu.async_copy(fbuf0.at[b], asum0.at[didx.at[b]], asem[b], add=True)
        pltpu.async_copy(fbuf1.at[b], asum1.at[didx.at[b]], asem[b], add=True)

    def wait_asum(b):
        pltpu.make_async_copy(fbuf0.at[b], asum0.at[didx.at[b]],
                              asem[b]).wait()
        pltpu.make_async_copy(fbuf1.at[b], asum1.at[didx.at[b]],
                              asem[b]).wait()

    def start_gather(b):
        pltpu.async_copy(xs_hbm.at[gidx.at[b]], gbuf.at[b], gsem[b])

    def wait_gather(b):
        pltpu.make_async_copy(xs_hbm.at[gidx.at[0]], gbuf.at[b],
                              gsem[b]).wait()

    def start_scatter(sb, b):
        pltpu.async_copy(sbuf.at[sb], out_acc.at[didx.at[b]], ssem[sb],
                         add=True)

    def wait_scatter(sb):
        pltpu.make_async_copy(sbuf.at[sb], out_acc.at[didx.at[0]],
                              ssem[sb]).wait()

    iota2 = lax.iota(jnp.int32, LANES) * 2

    def scale(b, sb):
        bv = jnp.full((LANES,), b, jnp.int32)
        sbv = jnp.full((LANES,), sb, jnp.int32)

        @plsc.parallel_loop(0, K, 1, unroll=4)
        def _scale(j):
            jv = jnp.full((LANES,), j, jnp.int32)
            s0 = plsc.load_gather(fbuf0, [bv, jv])
            s1 = plsc.load_gather(fbuf1, [bv, jv])
            for g in range(4):
                w = plsc.bitcast(gbuf[b, j, pl.ds(g * LANES, LANES)],
                                 jnp.bfloat16)
                ev, od = plsc.unpack(w, format=plsc.PackFormat.INTERLEAVED,
                                     preferred_element_type=jnp.float32)
                sc = s0 if g < 2 else s1
                plsc.store_scatter(sbuf, [sbv, jv, iota2 + (g * 32)],
                                   ev * sc)
                plsc.store_scatter(sbuf, [sbv, jv, iota2 + (g * 32 + 1)],
                                   od * sc)

    def prepare(group, ch0):
        for i, b in enumerate(group):
            wait_idx(b)
            prep_gidx(b)
            start_fgather(b)
            start_gather(b)
        for b in group:
            wait_fgather(b)
            start_asum(b)

    def consume(group, ch0, nxt):
        for i, b in enumerate(group):
            wait_gather(b)
            scale(b, i)
            start_scatter(i, b)
        for i, b in enumerate(group):
            wait_scatter(i)
            wait_asum(b)
            if nxt is not None:
                start_idx(nxt + i, b)

    G0 = (0, 1)
    G1 = (2, 3)
    NG = len(G0)

    for b in range(NBUF):
        start_idx(jnp.int32(b), b)
    prepare(G0, jnp.int32(0))

    @pl.loop(0, NCH // NBUF - 1)
    def _main(t):
        cb = t * NBUF
        prepare(G1, cb + NG)
        consume(G0, cb, cb + NBUF)
        prepare(G0, cb + NBUF)
        consume(G1, cb + NG, cb + NBUF + NG)

    lcb = jnp.int32(NCH - NBUF)
    prepare(G1, lcb + NG)
    consume(G0, lcb, None)
    consume(G1, lcb + NG, None)
    plsc.subcore_barrier()

    obase = c * NPAD + nbase
    for k in range(NPT // K):
        n0 = nbase + k * K
        pltpu.sync_copy(out_acc.at[pl.ds(n0, K)], sbuf.at[0])
        pltpu.sync_copy(asum0.at[pl.ds(n0, K)], abuf0)
        pltpu.sync_copy(asum1.at[pl.ds(n0, K)], abuf1)

        @pl.loop(0, K)
        def _norm(j):
            jv = jnp.full((LANES,), j, jnp.int32)
            s0 = 1.0 / jnp.maximum(plsc.load_gather(abuf0, [jv]), 1e-10)
            s1 = 1.0 / jnp.maximum(plsc.load_gather(abuf1, [jv]), 1e-10)
            for r in range(8):
                sl = sbuf[0, j, pl.ds(r * LANES, LANES)]
                sbuf[0, j, pl.ds(r * LANES, LANES)] = sl * (s0 if r < 4 else s1)

        pltpu.sync_copy(sbuf.at[0], out_hbm.at[pl.ds(obase + k * K, K)])


def _sc_aggregate(xs, tabs, srcp, dstp):
    mesh = plsc.VectorSubcoreMesh(core_axis_name="c", subcore_axis_name="s")
    return pl.kernel(
        _sc_body,
        out_type=jax.ShapeDtypeStruct((NC * NPAD, HALF), jnp.float32),
        mesh=mesh,
        compiler_params=pltpu.CompilerParams(needs_layout_passes=False,
                                             use_tc_tiling_on_sc=False),
        scratch_types=[
            pltpu.VMEM((NBUF, K), jnp.int32),
            pltpu.VMEM((NBUF, K), jnp.int32),
            pltpu.VMEM((NBUF, K), jnp.int32),
            pltpu.VMEM((NBUF, K), jnp.int32),
            pltpu.VMEM((NBUF, K), jnp.float32),
            pltpu.VMEM((NBUF, K), jnp.float32),
            pltpu.VMEM((NBUF, K, HALF // 2), jnp.int32),
            pltpu.VMEM((2, K, HALF), jnp.float32),
            pltpu.VMEM((K,), jnp.float32),
            pltpu.VMEM((K,), jnp.float32),
            pltpu.VMEM_SHARED((NPAD, HALF), jnp.float32),
            pltpu.VMEM_SHARED((NPAD,), jnp.float32),
            pltpu.VMEM_SHARED((NPAD,), jnp.float32),
            pltpu.VMEM_SHARED((NPAD,), jnp.float32),
            pltpu.VMEM_SHARED((NPAD,), jnp.float32),
        ] + [pltpu.SemaphoreType.DMA] * 20,
    )(xs, tabs, srcp, dstp)


def kernel(x, edge_index, att):
    x = x.astype(jnp.float32)
    att = att.astype(jnp.float32)
    src = edge_index[0].astype(jnp.int32)
    dst = edge_index[1].astype(jnp.int32)

    f, xsb = _node_tables(x, att)
    fpad = jnp.pad(f, ((0, NPAD - N_NODES), (0, 0)))
    tabs = fpad.reshape(NPAD, 2, 2).transpose(1, 2, 0)
    xs = lax.bitcast_convert_type(
        xsb.reshape(NC * N_NODES, HALF // 2, 2), jnp.int32)
    pad = E_PAD - N_EDGES
    srcp = jnp.pad(src, (0, pad), constant_values=N_NODES)
    dstp = jnp.pad(dst, (0, pad), constant_values=N_NODES)
    out2 = _sc_aggregate(xs, tabs, srcp, dstp)
    return (out2.reshape(NC, NPAD, HALF)[:, :N_NODES, :].transpose(1, 0, 2)
            .reshape(N_NODES, CHANNELS))
